# baseline (speedup 1.0000x reference)
import jax
import jax.numpy as jnp
from jax.experimental import pallas as pl
from jax.experimental.pallas import tpu as pltpu

E = 8
H = 1024
I = 768
T = 2048

SUB = 4
TS = T // SUB


def _moe_body(x_ref, rwt_ref, w1_ref, w2_ref, out_ref):
    e = pl.program_id(0)
    w1 = w1_ref[0].astype(jnp.bfloat16)
    w2 = w2_ref[0].astype(jnp.bfloat16)
    for i in range(SUB):
        sl = slice(i * TS, (i + 1) * TS)
        xb = x_ref[sl, :]
        gu = jnp.dot(xb, w1, preferred_element_type=jnp.float32)
        gate = gu[:, :I]
        up = gu[:, I:]
        inter = (up * (gate * jax.nn.sigmoid(gate))).astype(jnp.bfloat16)
        y = jnp.dot(inter, w2, preferred_element_type=jnp.float32)
        y = y * rwt_ref[0, 0, sl][:, None]

        @pl.when(e == 0)
        def _init():
            out_ref[sl, :] = y

        @pl.when(e != 0)
        def _acc():
            out_ref[sl, :] += y


def kernel(hidden_states, routing_weights, router_indices, gate_up_proj, down_proj):
    del router_indices
    x = hidden_states.reshape(T, H).astype(jnp.bfloat16)
    rwt = routing_weights.T.reshape(E, 1, T)

    out = pl.pallas_call(
        _moe_body,
        grid=(E,),
        in_specs=[
            pl.BlockSpec((T, H), lambda e: (0, 0)),
            pl.BlockSpec((1, 1, T), lambda e: (e, 0, 0)),
            pl.BlockSpec((1, H, 2 * I), lambda e: (e, 0, 0)),
            pl.BlockSpec((1, I, H), lambda e: (e, 0, 0)),
        ],
        out_specs=pl.BlockSpec((T, H), lambda e: (0, 0)),
        out_shape=jax.ShapeDtypeStruct((T, H), jnp.float32),
        compiler_params=pltpu.CompilerParams(
            dimension_semantics=("arbitrary",),
        ),
    )(x, rwt, gate_up_proj, down_proj)
    return out.reshape(T, 1, H)


# speedup vs baseline: 1.0317x; 1.0317x over previous
import jax
import jax.numpy as jnp
from jax.experimental import pallas as pl
from jax.experimental.pallas import tpu as pltpu

E = 8
H = 1024
I = 768
T = 2048

SUB = 2
TS = T // SUB


def _moe_body(x_ref, rwt_ref, w1_ref, w2_ref, out_ref):
    e = pl.program_id(0)
    w1 = w1_ref[0].astype(jnp.bfloat16)
    w2 = w2_ref[0].astype(jnp.bfloat16)
    for i in range(SUB):
        sl = slice(i * TS, (i + 1) * TS)
        xb = x_ref[sl, :]
        gu = jnp.dot(xb, w1, preferred_element_type=jnp.float32)
        gate = gu[:, :I]
        up = gu[:, I:]
        inter = (up * (gate * jax.nn.sigmoid(gate))).astype(jnp.bfloat16)
        y = jnp.dot(inter, w2, preferred_element_type=jnp.float32)
        y = y * rwt_ref[0, 0, sl][:, None]

        @pl.when(e == 0)
        def _init():
            out_ref[sl, :] = y

        @pl.when(e != 0)
        def _acc():
            out_ref[sl, :] += y


def kernel(hidden_states, routing_weights, router_indices, gate_up_proj, down_proj):
    del router_indices
    x = hidden_states.reshape(T, H).astype(jnp.bfloat16)
    rwt = routing_weights.T.reshape(E, 1, T)

    out = pl.pallas_call(
        _moe_body,
        grid=(E,),
        in_specs=[
            pl.BlockSpec((T, H), lambda e: (0, 0)),
            pl.BlockSpec((1, 1, T), lambda e: (e, 0, 0)),
            pl.BlockSpec((1, H, 2 * I), lambda e: (e, 0, 0)),
            pl.BlockSpec((1, I, H), lambda e: (e, 0, 0)),
        ],
        out_specs=pl.BlockSpec((T, H), lambda e: (0, 0)),
        out_shape=jax.ShapeDtypeStruct((T, H), jnp.float32),
        compiler_params=pltpu.CompilerParams(
            dimension_semantics=("arbitrary",),
        ),
    )(x, rwt, gate_up_proj, down_proj)
    return out.reshape(T, 1, H)


# speedup vs baseline: 1.2301x; 1.1923x over previous
import jax
import jax.numpy as jnp
from jax.experimental import pallas as pl
from jax.experimental.pallas import tpu as pltpu

E = 8
H = 1024
I = 768
T = 2048

SUB = 2
TS = T // SUB


def _moe_body(x_ref, rwt_ref, w1_ref, w2_ref, out_ref, xb_ref):
    e = pl.program_id(0)

    @pl.when(e == 0)
    def _cast_x():
        xb_ref[...] = x_ref[...].astype(jnp.bfloat16)

    w1 = w1_ref[0].astype(jnp.bfloat16)
    w2 = w2_ref[0].astype(jnp.bfloat16)
    for i in range(SUB):
        sl = slice(i * TS, (i + 1) * TS)
        xb = xb_ref[sl, :]
        gu = jnp.dot(xb, w1, preferred_element_type=jnp.float32)
        gate = gu[:, :I]
        up = gu[:, I:]
        inter = (up * (gate * jax.nn.sigmoid(gate))).astype(jnp.bfloat16)
        y = jnp.dot(inter, w2, preferred_element_type=jnp.float32)
        y = y * rwt_ref[0, 0, sl][:, None]

        @pl.when(e == 0)
        def _init():
            out_ref[sl, 0, :] = y

        @pl.when(e != 0)
        def _acc():
            out_ref[sl, 0, :] += y


def kernel(hidden_states, routing_weights, router_indices, gate_up_proj, down_proj):
    del router_indices
    x = hidden_states.reshape(T, H)
    rwt = routing_weights.T.reshape(E, 1, T)

    return pl.pallas_call(
        _moe_body,
        grid=(E,),
        in_specs=[
            pl.BlockSpec((T, H), lambda e: (0, 0)),
            pl.BlockSpec((1, 1, T), lambda e: (e, 0, 0)),
            pl.BlockSpec((1, H, 2 * I), lambda e: (e, 0, 0)),
            pl.BlockSpec((1, I, H), lambda e: (e, 0, 0)),
        ],
        out_specs=pl.BlockSpec((T, 1, H), lambda e: (0, 0, 0)),
        out_shape=jax.ShapeDtypeStruct((T, 1, H), jnp.float32),
        scratch_shapes=[pltpu.VMEM((T, H), jnp.bfloat16)],
        compiler_params=pltpu.CompilerParams(
            dimension_semantics=("arbitrary",),
        ),
    )(x, rwt, gate_up_proj, down_proj)
